# baseline (speedup 1.0000x reference)
import functools

import jax
import jax.numpy as jnp
from jax import lax
from jax.experimental import pallas as pl
from jax.experimental.pallas import tpu as pltpu
from jax.experimental.pallas import tpu_sc as plsc

N = 100000
E = 3200000
C = 16

NW = 32
NBIN = 32
BIN_N = N // NBIN
LCAP = 320
CAP = 16 * LCAP
BLK = 128
NBLK = CAP // BLK
RING = 4
NSUP = NBLK // RING
NSUP_S = NBLK // 4
ECAP = NW * NBIN * CAP
N_PAD = 102400
ACC_ROWS = 3136
LOC_ROWS = 3136
CHUNK = E // NW
QUINT = 10000
NQ = CHUNK // QUINT
BPP = 8
NPASS = NBIN // BPP

_MESH = plsc.VectorSubcoreMesh(
    core_axis_name="c", subcore_axis_name="s", num_cores=2, num_subcores=16)


def _wid():
    return lax.axis_index("s") * 2 + lax.axis_index("c")


def _bin_body(src_hbm, dst_hbm, srcp_hbm, dstp_hbm,
              in_src, in_dst, reg_src, reg_dst):
    wid = _wid()
    ebase = wid * CHUNK
    zeros16 = jnp.zeros((16,), jnp.int32)
    lanes = lax.iota(jnp.int32, 16) * LCAP
    for p in range(NPASS):

        def quint_body(q, offs, p=p):
            qs = ebase + q * QUINT
            pltpu.sync_copy(src_hbm.at[pl.ds(qs, QUINT)], in_src)
            pltpu.sync_copy(dst_hbm.at[pl.ds(qs, QUINT)], in_dst)

            def vec_body(v, offs):
                dvec = in_dst[pl.ds(v * 16, 16)]
                svec = in_src[pl.ds(v * 16, 16)]
                new_offs = []
                for bl in range(BPP):
                    lo = (p * BPP + bl) * BIN_N
                    off = offs[bl]
                    msk = (dvec >= lo) & (dvec < lo + BIN_N) & (off < LCAP)
                    pos = (bl * CAP + lanes) + off
                    plsc.store_scatter(reg_dst, [pos], dvec, mask=msk)
                    plsc.store_scatter(reg_src, [pos], svec, mask=msk)
                    new_offs.append(off + msk.astype(jnp.int32))
                return tuple(new_offs)

            return lax.fori_loop(0, QUINT // 16, vec_body, offs)

        offs = lax.fori_loop(0, NQ, quint_body,
                             tuple(zeros16 for _ in range(BPP)))

        for bl in range(BPP):
            gb = p * BPP + bl
            dummy = jnp.full((16,), (gb + 1) * BIN_N, jnp.int32)
            padsrc = jnp.full((16,), N, jnp.int32)

            def pad_body(i, off, bl=bl, dummy=dummy):
                msk = off < LCAP
                pos = (bl * CAP + lanes) + off
                plsc.store_scatter(reg_dst, [pos], dummy, mask=msk)
                plsc.store_scatter(reg_src, [pos], padsrc, mask=msk)
                return off + msk.astype(jnp.int32)

            lax.fori_loop(0, LCAP, pad_body, offs[bl])
            rbase = (wid * NBIN + gb) * CAP
            pltpu.sync_copy(reg_src.at[pl.ds(bl * CAP, CAP)],
                            srcp_hbm.at[pl.ds(rbase, CAP)])
            pltpu.sync_copy(reg_dst.at[pl.ds(bl * CAP, CAP)],
                            dstp_hbm.at[pl.ds(rbase, CAP)])


_bin_call = pl.kernel(
    _bin_body,
    out_type=(jax.ShapeDtypeStruct((ECAP,), jnp.int32),
              jax.ShapeDtypeStruct((ECAP,), jnp.int32)),
    mesh=_MESH,
    compiler_params=pltpu.CompilerParams(needs_layout_passes=False, use_tc_tiling_on_sc=False),
    scratch_types=[
        pltpu.VMEM((QUINT,), jnp.int32),
        pltpu.VMEM((QUINT,), jnp.int32),
        pltpu.VMEM((BPP * CAP,), jnp.int32),
        pltpu.VMEM((BPP * CAP,), jnp.int32),
    ],
)


def _bin2_body(srcp_hbm, dstp_hbm, srcp2_hbm, dstp2_hbm,
               in_src, in_dst, reg_src, reg_dst):
    db = _wid()
    zeros16 = jnp.zeros((16,), jnp.int32)
    lanes = lax.iota(jnp.int32, 16) * LCAP
    dummy = jnp.full((16,), (db + 1) * BIN_N, jnp.int32)
    for p in range(NPASS):

        def reg_body(w, offs, p=p):
            rbase = (w * NBIN + db) * CAP
            pltpu.sync_copy(srcp_hbm.at[pl.ds(rbase, CAP)], in_src)
            pltpu.sync_copy(dstp_hbm.at[pl.ds(rbase, CAP)], in_dst)

            def vec_body(v, offs):
                dvec = in_dst[pl.ds(v * 16, 16)]
                svec = in_src[pl.ds(v * 16, 16)]
                new_offs = []
                for bl in range(BPP):
                    lo = (p * BPP + bl) * BIN_N
                    off = offs[bl]
                    msk = (svec >= lo) & (svec < lo + BIN_N) & (off < LCAP)
                    pos = (bl * CAP + lanes) + off
                    plsc.store_scatter(reg_dst, [pos], dvec, mask=msk)
                    plsc.store_scatter(reg_src, [pos], svec, mask=msk)
                    new_offs.append(off + msk.astype(jnp.int32))
                return tuple(new_offs)

            return lax.fori_loop(0, CAP // 16, vec_body, offs)

        offs = lax.fori_loop(0, NBIN, reg_body,
                             tuple(zeros16 for _ in range(BPP)))

        for bl in range(BPP):
            sb = p * BPP + bl
            padsrc = jnp.full((16,), sb * BIN_N, jnp.int32)

            def pad_body(i, off, bl=bl, padsrc=padsrc):
                msk = off < LCAP
                pos = (bl * CAP + lanes) + off
                plsc.store_scatter(reg_dst, [pos], dummy, mask=msk)
                plsc.store_scatter(reg_src, [pos], padsrc, mask=msk)
                return off + msk.astype(jnp.int32)

            lax.fori_loop(0, LCAP, pad_body, offs[bl])
            rbase2 = (sb * NBIN + db) * CAP
            pltpu.sync_copy(reg_src.at[pl.ds(bl * CAP, CAP)],
                            srcp2_hbm.at[pl.ds(rbase2, CAP)])
            pltpu.sync_copy(reg_dst.at[pl.ds(bl * CAP, CAP)],
                            dstp2_hbm.at[pl.ds(rbase2, CAP)])


_bin2_call = pl.kernel(
    _bin2_body,
    out_type=(jax.ShapeDtypeStruct((ECAP,), jnp.int32),
              jax.ShapeDtypeStruct((ECAP,), jnp.int32)),
    mesh=_MESH,
    compiler_params=pltpu.CompilerParams(needs_layout_passes=False, use_tc_tiling_on_sc=False),
    scratch_types=[
        pltpu.VMEM((CAP,), jnp.int32),
        pltpu.VMEM((CAP,), jnp.int32),
        pltpu.VMEM((BPP * CAP,), jnp.int32),
        pltpu.VMEM((BPP * CAP,), jnp.int32),
    ],
)


def _asm_body(h_hbm, srcp2_hbm, dstp2_hbm, t_hbm,
              hs, hd, idx_s, idx_d, tbuf, so0, so1):
    sb = _wid()
    so = [so0, so1]
    pltpu.sync_copy(h_hbm.at[pl.ds(sb * BIN_N, LOC_ROWS)], hs)

    def drain(buf, sem):
        pltpu.make_async_copy(t_hbm.at[pl.ds(0, BLK // 4)], buf, sem).wait()

    def region_body(db, _):
        rbase = (sb * NBIN + db) * CAP
        pltpu.sync_copy(h_hbm.at[pl.ds(db * BIN_N, LOC_ROWS)], hd)
        pltpu.sync_copy(srcp2_hbm.at[pl.ds(rbase, CAP)], idx_s)
        pltpu.sync_copy(dstp2_hbm.at[pl.ds(rbase, CAP)], idx_d)
        sbase = sb * BIN_N
        dbase = db * BIN_N

        def blk_body(k2, _):
            for pp in range(2):
                k = k2 * 2 + pp

                @pl.when((k >= 2) | (db > 0))
                def _(pp=pp):
                    drain(tbuf.at[pp], so[pp])

                def grp_body(jg, _, pp=pp, k=k):
                    e0 = k * BLK + jg * 16
                    svec = idx_s[pl.ds(e0, 16)] - sbase
                    dvec = idx_d[pl.ds(e0, 16)] - dbase
                    for i in range(16):
                        hi = hd[dvec[i]]
                        hj = hs[svec[i]]
                        r4 = jg * 4 + i // 4
                        c4 = (i % 4) * 2 * C
                        tbuf[pp, r4, pl.ds(c4, C)] = hi
                        tbuf[pp, r4, pl.ds(c4 + C, C)] = hj - hi
                    return 0

                lax.fori_loop(0, BLK // 16, grp_body, 0)
                pltpu.async_copy(
                    tbuf.at[pp],
                    t_hbm.at[pl.ds((rbase + k * BLK) // 4, BLK // 4)], so[pp])
            return 0

        lax.fori_loop(0, NBLK // 2, blk_body, 0)
        return 0

    lax.fori_loop(0, NBIN, region_body, 0)
    for pp in range(2):
        drain(tbuf.at[pp], so[pp])


_asm_call = pl.kernel(
    _asm_body,
    out_type=jax.ShapeDtypeStruct((ECAP // 4, 8 * C), jnp.float32),
    mesh=_MESH,
    compiler_params=pltpu.CompilerParams(needs_layout_passes=False, use_tc_tiling_on_sc=False),
    scratch_types=(
        [pltpu.VMEM((LOC_ROWS, C), jnp.float32)] * 2
        + [pltpu.VMEM((CAP,), jnp.int32)] * 2
        + [pltpu.VMEM((2, BLK // 4, 8 * C), jnp.float32)]
        + [pltpu.SemaphoreType.DMA] * 2
    ),
)


def _scatter_body(m_hbm, dstp_hbm, h_hbm, acc, acc_b, dstv, mbuf,
                  sm0, sm1, sm2, sm3):
    tid = _wid()
    base_node = tid * BIN_N
    sm = [sm0, sm1, sm2, sm3]

    def z_body(i, _):
        acc[i] = jnp.zeros((C,), jnp.float32)
        acc_b[i] = jnp.zeros((C,), jnp.float32)
        return 0

    lax.fori_loop(0, ACC_ROWS, z_body, 0)

    def region_body(r, _):
        rbase = (r * NBIN + tid) * CAP
        pltpu.sync_copy(dstp_hbm.at[pl.ds(rbase, CAP)], dstv)
        for pp in range(4):
            pltpu.async_copy(
                m_hbm.at[pl.ds((rbase + pp * BLK) // 4, BLK // 4)],
                mbuf.at[pp], sm[pp])

        def super_body(s, _):
            for pp in range(4):
                b = s * 4 + pp
                pltpu.make_async_copy(
                    m_hbm.at[pl.ds(0, BLK // 4)], mbuf.at[pp], sm[pp]).wait()

                def grp_body(jg, _, pp=pp):
                    dvec = dstv[pl.ds(b * BLK + jg * 16, 16)] - base_node
                    for i in range(16):
                        d = dvec[i]
                        row = mbuf[pp, jg * 4 + i // 4, pl.ds((i % 4) * C, C)]
                        a = acc if i % 2 == 0 else acc_b
                        a[d] = jnp.maximum(a[d], row)
                    return 0

                lax.fori_loop(0, BLK // 16, grp_body, 0)

                @pl.when(s < NSUP_S - 1)
                def _(pp=pp, b=b):
                    pltpu.async_copy(
                        m_hbm.at[pl.ds((rbase + (b + 4) * BLK) // 4, BLK // 4)],
                        mbuf.at[pp], sm[pp])

            return 0

        lax.fori_loop(0, NSUP_S, super_body, 0)
        return 0

    lax.fori_loop(0, NBIN, region_body, 0)

    def mrg_body(i, _):
        acc[i] = jnp.maximum(acc[i], acc_b[i])
        return 0

    lax.fori_loop(0, BIN_N, mrg_body, 0)
    pltpu.sync_copy(acc.at[pl.ds(0, BIN_N)],
                    h_hbm.at[pl.ds(base_node, BIN_N)])


_scatter_call = pl.kernel(
    _scatter_body,
    out_type=jax.ShapeDtypeStruct((N_PAD, C), jnp.float32),
    mesh=_MESH,
    compiler_params=pltpu.CompilerParams(needs_layout_passes=False, use_tc_tiling_on_sc=False),
    scratch_types=(
        [pltpu.VMEM((ACC_ROWS, C), jnp.float32),
         pltpu.VMEM((ACC_ROWS, C), jnp.float32),
         pltpu.VMEM((CAP,), jnp.int32),
         pltpu.VMEM((4, BLK // 4, 8 * C), jnp.float32)]
        + [pltpu.SemaphoreType.DMA] * 4
    ),
)


_PREC = jax.lax.Precision.HIGHEST


def _mlp_body(t_ref, wa_ref, ba_ref, wb_ref, bb_ref, m_ref):
    z = jnp.maximum(t_ref[...] @ wa_ref[...] + ba_ref[...], 0.0)
    m = z @ wb_ref[...] + bb_ref[...]
    m_ref[...] = jnp.concatenate([m, jnp.zeros_like(m)], axis=1)


def _mlp_stage(T, Wa, ba, Wb, bb):
    eye4 = jnp.eye(4, dtype=jnp.float32)
    BR = 2048
    return pl.pallas_call(
        _mlp_body,
        grid=(ECAP // 4 // BR,),
        in_specs=[
            pl.BlockSpec((BR, 8 * C), lambda i: (i, 0)),
            pl.BlockSpec((8 * C, 4 * C), lambda i: (0, 0)),
            pl.BlockSpec((1, 4 * C), lambda i: (0, 0)),
            pl.BlockSpec((4 * C, 4 * C), lambda i: (0, 0)),
            pl.BlockSpec((1, 4 * C), lambda i: (0, 0)),
        ],
        out_specs=pl.BlockSpec((BR, 8 * C), lambda i: (i, 0)),
        out_shape=jax.ShapeDtypeStruct((ECAP // 4, 8 * C), jnp.float32),
    )(T, jnp.kron(eye4, Wa), jnp.tile(ba, 4).reshape(1, 4 * C),
      jnp.kron(eye4, Wb), jnp.tile(bb, 4).reshape(1, 4 * C))


def _final_body(x2_ref, x4_ref, x6_ref, wf_ref, bf_ref, o_ref):
    x7 = jnp.concatenate([x2_ref[...], x4_ref[...], x6_ref[...]], axis=1)
    cols = jnp.arange(48)[:, None]
    ks = jnp.arange(C)[None, :]
    sel = [(cols == 3 * ks + c).astype(jnp.float32) for c in range(3)]
    x8 = jnp.maximum(jnp.maximum(jnp.dot(x7, sel[0], precision=_PREC),
                                 jnp.dot(x7, sel[1], precision=_PREC)),
                     jnp.dot(x7, sel[2], precision=_PREC))
    o_ref[...] = (x8 @ wf_ref[...])[:, 0:1] + bf_ref[0]


def _final_stage(x2, x4, x6, Wf, bf):
    BR = 2000
    return pl.pallas_call(
        _final_body,
        grid=(N // BR,),
        in_specs=[
            pl.BlockSpec((BR, C), lambda i: (i, 0)),
            pl.BlockSpec((BR, C), lambda i: (i, 0)),
            pl.BlockSpec((BR, C), lambda i: (i, 0)),
            pl.BlockSpec((C, 128), lambda i: (0, 0)),
            pl.BlockSpec(memory_space=pltpu.SMEM),
        ],
        out_specs=pl.BlockSpec((BR, 1), lambda i: (i, 0)),
        out_shape=jax.ShapeDtypeStruct((N, 1), jnp.float32),
    )(x2, x4, x6, jnp.pad(Wf, ((0, 0), (0, 127))), bf)


def kernel(x, edge_index, W1a, b1a, W1b, b1b, W2a, b2a, W2b, b2b,
           W3a, b3a, W3b, b3b, Wf, bf):
    src = edge_index[0]
    dst = edge_index[1]
    srcp1, dstp1 = _bin_call(src, dst)
    srcp2, dstp2 = _bin2_call(srcp1, dstp1)

    W1a_pad = (jnp.zeros((2 * C, C), jnp.float32)
               .at[0:3].set(W1a[0:3]).at[C:C + 3].set(W1a[3:6]))
    x_pad = jnp.zeros((N_PAD, C), jnp.float32).at[:N, :3].set(x)

    layers = [
        (W1a_pad, b1a, W1b, b1b),
        (W2a, b2a, W2b, b2b),
        (W3a, b3a, W3b, b3b),
    ]
    h = x_pad
    hs = []
    for (Wa, ba, Wb, bb) in layers:
        T = _asm_call(h, srcp2, dstp2)
        m = _mlp_stage(T, Wa, ba, Wb, bb)
        h = _scatter_call(m, dstp2)
        hs.append(h)

    return _final_stage(hs[0], hs[1], hs[2], Wf, bf)


# speedup vs baseline: 1.0042x; 1.0042x over previous
import functools

import jax
import jax.numpy as jnp
from jax import lax
from jax.experimental import pallas as pl
from jax.experimental.pallas import tpu as pltpu
from jax.experimental.pallas import tpu_sc as plsc

N = 100000
E = 3200000
C = 16

NW = 32
NBIN = 32
BIN_N = N // NBIN
LCAP = 320
CAP = 16 * LCAP
BLK = 128
NBLK = CAP // BLK
RING = 4
NSUP = NBLK // RING
NSUP_S = NBLK // 4
ECAP = NW * NBIN * CAP
N_PAD = 102400
ACC_ROWS = 3136
LOC_ROWS = 3136
CHUNK = E // NW
QUINT = 10000
NQ = CHUNK // QUINT
BPP = 8
NPASS = NBIN // BPP

_MESH = plsc.VectorSubcoreMesh(
    core_axis_name="c", subcore_axis_name="s", num_cores=2, num_subcores=16)


def _wid():
    return lax.axis_index("s") * 2 + lax.axis_index("c")


def _bin_body(src_hbm, dst_hbm, srcp_hbm, dstp_hbm,
              in_src, in_dst, reg_src, reg_dst):
    wid = _wid()
    ebase = wid * CHUNK
    zeros16 = jnp.zeros((16,), jnp.int32)
    lanes = lax.iota(jnp.int32, 16) * LCAP
    for p in range(NPASS):

        def quint_body(q, offs, p=p):
            qs = ebase + q * QUINT
            pltpu.sync_copy(src_hbm.at[pl.ds(qs, QUINT)], in_src)
            pltpu.sync_copy(dst_hbm.at[pl.ds(qs, QUINT)], in_dst)

            def vec_body(v, offs):
                dvec = in_dst[pl.ds(v * 16, 16)]
                svec = in_src[pl.ds(v * 16, 16)]
                new_offs = []
                for bl in range(BPP):
                    lo = (p * BPP + bl) * BIN_N
                    off = offs[bl]
                    msk = (dvec >= lo) & (dvec < lo + BIN_N) & (off < LCAP)
                    pos = (bl * CAP + lanes) + off
                    plsc.store_scatter(reg_dst, [pos], dvec, mask=msk)
                    plsc.store_scatter(reg_src, [pos], svec, mask=msk)
                    new_offs.append(off + msk.astype(jnp.int32))
                return tuple(new_offs)

            return lax.fori_loop(0, QUINT // 16, vec_body, offs)

        offs = lax.fori_loop(0, NQ, quint_body,
                             tuple(zeros16 for _ in range(BPP)))

        for bl in range(BPP):
            gb = p * BPP + bl
            dummy = jnp.full((16,), (gb + 1) * BIN_N, jnp.int32)
            padsrc = jnp.full((16,), N, jnp.int32)

            def pad_body(i, off, bl=bl, dummy=dummy):
                msk = off < LCAP
                pos = (bl * CAP + lanes) + off
                plsc.store_scatter(reg_dst, [pos], dummy, mask=msk)
                plsc.store_scatter(reg_src, [pos], padsrc, mask=msk)
                return off + msk.astype(jnp.int32)

            lax.fori_loop(0, LCAP, pad_body, offs[bl])
            rbase = (wid * NBIN + gb) * CAP
            pltpu.sync_copy(reg_src.at[pl.ds(bl * CAP, CAP)],
                            srcp_hbm.at[pl.ds(rbase, CAP)])
            pltpu.sync_copy(reg_dst.at[pl.ds(bl * CAP, CAP)],
                            dstp_hbm.at[pl.ds(rbase, CAP)])


_bin_call = pl.kernel(
    _bin_body,
    out_type=(jax.ShapeDtypeStruct((ECAP,), jnp.int32),
              jax.ShapeDtypeStruct((ECAP,), jnp.int32)),
    mesh=_MESH,
    compiler_params=pltpu.CompilerParams(needs_layout_passes=False, use_tc_tiling_on_sc=False),
    scratch_types=[
        pltpu.VMEM((QUINT,), jnp.int32),
        pltpu.VMEM((QUINT,), jnp.int32),
        pltpu.VMEM((BPP * CAP,), jnp.int32),
        pltpu.VMEM((BPP * CAP,), jnp.int32),
    ],
)


def _bin2_body(srcp_hbm, dstp_hbm, srcp2_hbm, dstp2_hbm,
               in_src, in_dst, reg_src, reg_dst):
    db = _wid()
    zeros16 = jnp.zeros((16,), jnp.int32)
    lanes = lax.iota(jnp.int32, 16) * LCAP
    dummy = jnp.full((16,), (db + 1) * BIN_N, jnp.int32)
    for p in range(NPASS):

        def reg_body(w, offs, p=p):
            rbase = (w * NBIN + db) * CAP
            pltpu.sync_copy(srcp_hbm.at[pl.ds(rbase, CAP)], in_src)
            pltpu.sync_copy(dstp_hbm.at[pl.ds(rbase, CAP)], in_dst)

            def vec_body(v, offs):
                dvec = in_dst[pl.ds(v * 16, 16)]
                svec = in_src[pl.ds(v * 16, 16)]
                new_offs = []
                for bl in range(BPP):
                    lo = (p * BPP + bl) * BIN_N
                    off = offs[bl]
                    msk = (svec >= lo) & (svec < lo + BIN_N) & (off < LCAP)
                    pos = (bl * CAP + lanes) + off
                    plsc.store_scatter(reg_dst, [pos], dvec, mask=msk)
                    plsc.store_scatter(reg_src, [pos], svec, mask=msk)
                    new_offs.append(off + msk.astype(jnp.int32))
                return tuple(new_offs)

            return lax.fori_loop(0, CAP // 16, vec_body, offs)

        offs = lax.fori_loop(0, NBIN, reg_body,
                             tuple(zeros16 for _ in range(BPP)))

        for bl in range(BPP):
            sb = p * BPP + bl
            padsrc = jnp.full((16,), sb * BIN_N, jnp.int32)

            def pad_body(i, off, bl=bl, padsrc=padsrc):
                msk = off < LCAP
                pos = (bl * CAP + lanes) + off
                plsc.store_scatter(reg_dst, [pos], dummy, mask=msk)
                plsc.store_scatter(reg_src, [pos], padsrc, mask=msk)
                return off + msk.astype(jnp.int32)

            lax.fori_loop(0, LCAP, pad_body, offs[bl])
            rbase2 = (sb * NBIN + db) * CAP
            pltpu.sync_copy(reg_src.at[pl.ds(bl * CAP, CAP)],
                            srcp2_hbm.at[pl.ds(rbase2, CAP)])
            pltpu.sync_copy(reg_dst.at[pl.ds(bl * CAP, CAP)],
                            dstp2_hbm.at[pl.ds(rbase2, CAP)])


_bin2_call = pl.kernel(
    _bin2_body,
    out_type=(jax.ShapeDtypeStruct((ECAP,), jnp.int32),
              jax.ShapeDtypeStruct((ECAP,), jnp.int32)),
    mesh=_MESH,
    compiler_params=pltpu.CompilerParams(needs_layout_passes=False, use_tc_tiling_on_sc=False),
    scratch_types=[
        pltpu.VMEM((CAP,), jnp.int32),
        pltpu.VMEM((CAP,), jnp.int32),
        pltpu.VMEM((BPP * CAP,), jnp.int32),
        pltpu.VMEM((BPP * CAP,), jnp.int32),
    ],
)


def _asm_body(h_hbm, srcp2_hbm, dstp2_hbm, t_hbm,
              hs, hd, idx_s, idx_d, tbuf, so0, so1):
    sb = _wid()
    so = [so0, so1]
    pltpu.sync_copy(h_hbm.at[pl.ds(sb * BIN_N, LOC_ROWS)], hs)

    def drain(buf, sem):
        pltpu.make_async_copy(t_hbm.at[pl.ds(0, BLK // 4)], buf, sem).wait()

    def region_body(db, _):
        rbase = (sb * NBIN + db) * CAP
        pltpu.sync_copy(h_hbm.at[pl.ds(db * BIN_N, LOC_ROWS)], hd)
        pltpu.sync_copy(srcp2_hbm.at[pl.ds(rbase, CAP)], idx_s)
        pltpu.sync_copy(dstp2_hbm.at[pl.ds(rbase, CAP)], idx_d)
        sbase = sb * BIN_N
        dbase = db * BIN_N

        def blk_body(k2, _):
            for pp in range(2):
                k = k2 * 2 + pp

                @pl.when((k >= 2) | (db > 0))
                def _(pp=pp):
                    drain(tbuf.at[pp], so[pp])

                def grp_body(jg, _, pp=pp, k=k):
                    e0 = k * BLK + jg * 16
                    svec = idx_s[pl.ds(e0, 16)] - sbase
                    dvec = idx_d[pl.ds(e0, 16)] - dbase
                    for i in range(16):
                        hi = hd[dvec[i]]
                        hj = hs[svec[i]]
                        r4 = jg * 4 + i // 4
                        c4 = (i % 4) * 2 * C
                        tbuf[pp, r4, pl.ds(c4, C)] = hi
                        tbuf[pp, r4, pl.ds(c4 + C, C)] = hj - hi
                    return 0

                lax.fori_loop(0, BLK // 16, grp_body, 0)
                pltpu.async_copy(
                    tbuf.at[pp],
                    t_hbm.at[pl.ds((rbase + k * BLK) // 4, BLK // 4)], so[pp])
            return 0

        lax.fori_loop(0, NBLK // 2, blk_body, 0)
        return 0

    lax.fori_loop(0, NBIN, region_body, 0)
    for pp in range(2):
        drain(tbuf.at[pp], so[pp])


_asm_call = pl.kernel(
    _asm_body,
    out_type=jax.ShapeDtypeStruct((ECAP // 4, 8 * C), jnp.float32),
    mesh=_MESH,
    compiler_params=pltpu.CompilerParams(needs_layout_passes=False, use_tc_tiling_on_sc=False),
    scratch_types=(
        [pltpu.VMEM((LOC_ROWS, C), jnp.float32)] * 2
        + [pltpu.VMEM((CAP,), jnp.int32)] * 2
        + [pltpu.VMEM((2, BLK // 4, 8 * C), jnp.float32)]
        + [pltpu.SemaphoreType.DMA] * 2
    ),
)


def _scatter_body(m_hbm, dstp_hbm, h_hbm, acc, dstv, mbuf, sm0, sm1, sm2, sm3):
    tid = _wid()
    base_node = tid * BIN_N
    sm = [sm0, sm1, sm2, sm3]

    def z_body(i, _):
        acc[i] = jnp.zeros((C,), jnp.float32)
        return 0

    lax.fori_loop(0, ACC_ROWS, z_body, 0)

    def region_body(r, _):
        rbase = (r * NBIN + tid) * CAP
        pltpu.sync_copy(dstp_hbm.at[pl.ds(rbase, CAP)], dstv)
        for pp in range(4):
            pltpu.async_copy(
                m_hbm.at[pl.ds((rbase + pp * BLK) // 4, BLK // 4)],
                mbuf.at[pp], sm[pp])

        def super_body(s, _):
            for pp in range(4):
                b = s * 4 + pp
                pltpu.make_async_copy(
                    m_hbm.at[pl.ds(0, BLK // 4)], mbuf.at[pp], sm[pp]).wait()

                def grp_body(jg, _, pp=pp):
                    dvec = dstv[pl.ds(b * BLK + jg * 16, 16)] - base_node
                    for i in range(16):
                        d = dvec[i]
                        row = mbuf[pp, jg * 4 + i // 4, pl.ds((i % 4) * C, C)]
                        acc[d] = jnp.maximum(acc[d], row)
                    return 0

                lax.fori_loop(0, BLK // 16, grp_body, 0)

                @pl.when(s < NSUP_S - 1)
                def _(pp=pp, b=b):
                    pltpu.async_copy(
                        m_hbm.at[pl.ds((rbase + (b + 4) * BLK) // 4, BLK // 4)],
                        mbuf.at[pp], sm[pp])

            return 0

        lax.fori_loop(0, NSUP_S, super_body, 0)
        return 0

    lax.fori_loop(0, NBIN, region_body, 0)
    pltpu.sync_copy(acc.at[pl.ds(0, BIN_N)],
                    h_hbm.at[pl.ds(base_node, BIN_N)])


_scatter_call = pl.kernel(
    _scatter_body,
    out_type=jax.ShapeDtypeStruct((N_PAD, C), jnp.float32),
    mesh=_MESH,
    compiler_params=pltpu.CompilerParams(needs_layout_passes=False, use_tc_tiling_on_sc=False),
    scratch_types=(
        [pltpu.VMEM((ACC_ROWS, C), jnp.float32),
         pltpu.VMEM((CAP,), jnp.int32),
         pltpu.VMEM((4, BLK // 4, 8 * C), jnp.float32)]
        + [pltpu.SemaphoreType.DMA] * 4
    ),
)


_PREC = jax.lax.Precision.HIGHEST


def _mlp_body(t_ref, wa_ref, ba_ref, wb_ref, bb_ref, m_ref):
    z = jnp.maximum(t_ref[...] @ wa_ref[...] + ba_ref[...], 0.0)
    m = z @ wb_ref[...] + bb_ref[...]
    m_ref[...] = jnp.concatenate([m, jnp.zeros_like(m)], axis=1)


def _mlp_stage(T, Wa, ba, Wb, bb):
    eye4 = jnp.eye(4, dtype=jnp.float32)
    BR = 2048
    return pl.pallas_call(
        _mlp_body,
        grid=(ECAP // 4 // BR,),
        in_specs=[
            pl.BlockSpec((BR, 8 * C), lambda i: (i, 0)),
            pl.BlockSpec((8 * C, 4 * C), lambda i: (0, 0)),
            pl.BlockSpec((1, 4 * C), lambda i: (0, 0)),
            pl.BlockSpec((4 * C, 4 * C), lambda i: (0, 0)),
            pl.BlockSpec((1, 4 * C), lambda i: (0, 0)),
        ],
        out_specs=pl.BlockSpec((BR, 8 * C), lambda i: (i, 0)),
        out_shape=jax.ShapeDtypeStruct((ECAP // 4, 8 * C), jnp.float32),
    )(T, jnp.kron(eye4, Wa), jnp.tile(ba, 4).reshape(1, 4 * C),
      jnp.kron(eye4, Wb), jnp.tile(bb, 4).reshape(1, 4 * C))


def _final_body(x2_ref, x4_ref, x6_ref, wf_ref, bf_ref, o_ref):
    x7 = jnp.concatenate([x2_ref[...], x4_ref[...], x6_ref[...]], axis=1)
    cols = jnp.arange(48)[:, None]
    ks = jnp.arange(C)[None, :]
    sel = [(cols == 3 * ks + c).astype(jnp.float32) for c in range(3)]
    x8 = jnp.maximum(jnp.maximum(jnp.dot(x7, sel[0], precision=_PREC),
                                 jnp.dot(x7, sel[1], precision=_PREC)),
                     jnp.dot(x7, sel[2], precision=_PREC))
    o_ref[...] = (x8 @ wf_ref[...])[:, 0:1] + bf_ref[0]


def _final_stage(x2, x4, x6, Wf, bf):
    BR = 2000
    return pl.pallas_call(
        _final_body,
        grid=(N // BR,),
        in_specs=[
            pl.BlockSpec((BR, C), lambda i: (i, 0)),
            pl.BlockSpec((BR, C), lambda i: (i, 0)),
            pl.BlockSpec((BR, C), lambda i: (i, 0)),
            pl.BlockSpec((C, 128), lambda i: (0, 0)),
            pl.BlockSpec(memory_space=pltpu.SMEM),
        ],
        out_specs=pl.BlockSpec((BR, 1), lambda i: (i, 0)),
        out_shape=jax.ShapeDtypeStruct((N, 1), jnp.float32),
    )(x2, x4, x6, jnp.pad(Wf, ((0, 0), (0, 127))), bf)


def kernel(x, edge_index, W1a, b1a, W1b, b1b, W2a, b2a, W2b, b2b,
           W3a, b3a, W3b, b3b, Wf, bf):
    src = edge_index[0]
    dst = edge_index[1]
    srcp1, dstp1 = _bin_call(src, dst)
    srcp2, dstp2 = _bin2_call(srcp1, dstp1)

    W1a_pad = (jnp.zeros((2 * C, C), jnp.float32)
               .at[0:3].set(W1a[0:3]).at[C:C + 3].set(W1a[3:6]))
    x_pad = jnp.zeros((N_PAD, C), jnp.float32).at[:N, :3].set(x)

    layers = [
        (W1a_pad, b1a, W1b, b1b),
        (W2a, b2a, W2b, b2b),
        (W3a, b3a, W3b, b3b),
    ]
    h = x_pad
    hs = []
    for (Wa, ba, Wb, bb) in layers:
        T = _asm_call(h, srcp2, dstp2)
        m = _mlp_stage(T, Wa, ba, Wb, bb)
        h = _scatter_call(m, dstp2)
        hs.append(h)

    return _final_stage(hs[0], hs[1], hs[2], Wf, bf)
